# baseline (speedup 1.0000x reference)
import functools

import jax
import jax.numpy as jnp
import numpy as np
from jax import lax
from jax.experimental import pallas as pl
from jax.experimental.pallas import tpu as pltpu
from jax.experimental.pallas import tpu_sc as plsc

N_ROWS = 8192
N_FEAT = 1024
N_BINS = 16
NC = 2
NS = 16
NW = NC * NS
ROWS_PER_W = N_ROWS // NW
BLK = 32
NBLK = ROWS_PER_W // BLK
LANES = 16

_BIN_COUNTS = np.array(
    [900, 1200, 750, 1100, 980, 640, 1300, 870, 1020, 760, 1150, 690, 940,
     1080, 820, 1010], dtype=np.float64)


def _static_weights():
    mat_rows = []
    w_rows = []
    for gs in (2, 4, 8):
        g = N_BINS // gs
        sums = _BIN_COUNTS.reshape(g, gs).sum(axis=1)
        diffs = sums.max() - sums
        md = diffs.max() if diffs.max() != 0 else 1.0
        norm = (diffs / md).astype(np.float32)
        for i in range(g):
            row = np.zeros(N_BINS, dtype=np.float32)
            row[i * gs:(i + 1) * gs] = 1.0
            mat_rows.append(row)
            w_rows.append(np.exp(norm[i]) / np.float32(g))
    while len(mat_rows) < 16:
        mat_rows.append(np.zeros(N_BINS, dtype=np.float32))
        w_rows.append(np.float32(0.0))
    return np.stack(mat_rows), np.array(w_rows, dtype=np.float32).reshape(16, 1)


_GROUP_MAT, _GROUP_W = _static_weights()


def _sc_body(feat_hbm, lab_hbm, outsum, outsq, outcnt,
             rows_a, rows_b, lab_v, idx_v, acc_sum, acc_sq, acc_cnt,
             sem_a, sem_b):
    c = lax.axis_index("c")
    s = lax.axis_index("s")
    wid = c * NS + s
    base = wid * ROWS_PER_W
    zero16 = jnp.zeros((LANES,), jnp.float32)
    lane = lax.iota(jnp.int32, LANES)
    one_row = jnp.where(lane == 0, jnp.float32(1.0), jnp.float32(0.0))

    def zrow(i, _):
        for b in range(N_BINS):
            acc_sum[b, pl.ds(i * LANES, LANES)] = zero16
        return 0
    lax.fori_loop(0, N_FEAT // LANES, zrow, 0)
    for b in range(N_BINS):
        acc_sq[b, pl.ds(0, LANES)] = zero16
        acc_cnt[b, pl.ds(0, LANES)] = zero16

    pltpu.sync_copy(lab_hbm.at[pl.ds(base, ROWS_PER_W)], lab_v)
    for g in range(ROWS_PER_W // LANES):
        lv = lab_v[pl.ds(g * LANES, LANES)]
        bi = (lv * jnp.float32(N_BINS)).astype(jnp.int32)
        idx_v[pl.ds(g * LANES, LANES)] = jnp.clip(bi, 0, N_BINS - 1)

    def start(blk, buf, sem):
        pltpu.async_copy(feat_hbm.at[pl.ds(base + blk * BLK, BLK)], buf, sem)

    def wait(blk, buf, sem):
        pltpu.make_async_copy(feat_hbm.at[pl.ds(base + blk * BLK, BLK)],
                              buf, sem).wait()

    ILV = 8

    def accumulate(blk, buf):
        def rown(rr, _):
            r = rr * ILV
            bins = [idx_v[pl.ds(blk * BLK + r + i, LANES)][0]
                    for i in range(ILV)]
            sq = [zero16] * ILV
            for ch in range(N_FEAT // LANES):
                sl = pl.ds(ch * LANES, LANES)
                vs = [buf[r + i, sl] for i in range(ILV)]
                for i in range(ILV):
                    plsc.addupdate(acc_sum.at[bins[i], sl], vs[i])
                for i in range(ILV):
                    sq[i] = sq[i] + vs[i] * vs[i]
            for i in range(ILV):
                plsc.addupdate(acc_sq.at[bins[i], pl.ds(0, LANES)], sq[i])
                plsc.addupdate(acc_cnt.at[bins[i], pl.ds(0, LANES)], one_row)
            return 0

        lax.fori_loop(0, BLK // ILV, rown, 0)

    start(0, rows_a, sem_a)
    def pipe(h, _):
        b0 = h * 2
        start(b0 + 1, rows_b, sem_b)
        wait(b0, rows_a, sem_a)
        accumulate(b0, rows_a)

        @pl.when(b0 + 2 < NBLK)
        def _():
            start(b0 + 2, rows_a, sem_a)

        wait(b0 + 1, rows_b, sem_b)
        accumulate(b0 + 1, rows_b)
        return 0

    lax.fori_loop(0, NBLK // 2, pipe, 0)

    pltpu.sync_copy(acc_sum, outsum.at[wid])
    pltpu.sync_copy(acc_sq, outsq.at[wid])
    pltpu.sync_copy(acc_cnt, outcnt.at[wid])


def _make_sc_call():
    mesh = plsc.VectorSubcoreMesh(core_axis_name="c", subcore_axis_name="s")
    return pl.kernel(
        _sc_body,
        out_type=[
            jax.ShapeDtypeStruct((NW, N_BINS, N_FEAT), jnp.float32),
            jax.ShapeDtypeStruct((NW, N_BINS, LANES), jnp.float32),
            jax.ShapeDtypeStruct((NW, N_BINS, LANES), jnp.float32),
        ],
        mesh=mesh,
        scratch_types=[
            pltpu.VMEM((BLK, N_FEAT), jnp.float32),
            pltpu.VMEM((BLK, N_FEAT), jnp.float32),
            pltpu.VMEM((ROWS_PER_W,), jnp.float32),
            pltpu.VMEM((ROWS_PER_W + LANES,), jnp.int32),
            pltpu.VMEM((N_BINS, N_FEAT), jnp.float32),
            pltpu.VMEM((N_BINS, LANES), jnp.float32),
            pltpu.VMEM((N_BINS, LANES), jnp.float32),
            pltpu.SemaphoreType.DMA,
            pltpu.SemaphoreType.DMA,
        ],
    )


def _combine_body(sum_ref, sq_ref, cnt_ref, mat_ref, w_ref, out_ref):
    binsum = sum_ref[0]
    binsq = sq_ref[0]
    cnt = cnt_ref[0]
    for w in range(1, NW):
        binsum = binsum + sum_ref[w]
        binsq = binsq + sq_ref[w]
        cnt = cnt + cnt_ref[w]
    q = jnp.sum(binsq, axis=1, keepdims=True)
    c = cnt[:, 0:1]
    m = mat_ref[...]
    sg = jnp.dot(m, binsum, preferred_element_type=jnp.float32)
    qg = jnp.dot(m, q, preferred_element_type=jnp.float32)
    cg = jnp.maximum(jnp.dot(m, c, preferred_element_type=jnp.float32), 1.0)
    nrm = jnp.sum(sg * sg, axis=1, keepdims=True)
    l2 = qg / cg - nrm / (cg * cg)
    out_ref[0, 0] = jnp.sum(l2 * w_ref[...])


def kernel(features, labels):
    sc_call = _make_sc_call()
    psum, psq, pcnt = sc_call(features, labels)
    loss = pl.pallas_call(
        _combine_body,
        out_shape=jax.ShapeDtypeStruct((1, 1), jnp.float32),
        out_specs=pl.BlockSpec(memory_space=pltpu.SMEM),
    )(psum, psq, pcnt, jnp.asarray(_GROUP_MAT), jnp.asarray(_GROUP_W))
    return loss.reshape(1)


# speedup vs baseline: 1.0809x; 1.0809x over previous
import functools

import jax
import jax.numpy as jnp
import numpy as np
from jax import lax
from jax.experimental import pallas as pl
from jax.experimental.pallas import tpu as pltpu
from jax.experimental.pallas import tpu_sc as plsc

N_ROWS = 8192
N_FEAT = 1024
N_BINS = 16
NC = 2
NS = 16
NW = NC * NS
ROWS_PER_W = N_ROWS // NW
BLK = 32
NBLK = ROWS_PER_W // BLK
LANES = 16

_BIN_COUNTS = np.array(
    [900, 1200, 750, 1100, 980, 640, 1300, 870, 1020, 760, 1150, 690, 940,
     1080, 820, 1010], dtype=np.float64)


def _static_weights():
    mat_rows = []
    w_rows = []
    for gs in (2, 4, 8):
        g = N_BINS // gs
        sums = _BIN_COUNTS.reshape(g, gs).sum(axis=1)
        diffs = sums.max() - sums
        md = diffs.max() if diffs.max() != 0 else 1.0
        norm = (diffs / md).astype(np.float32)
        for i in range(g):
            row = np.zeros(N_BINS, dtype=np.float32)
            row[i * gs:(i + 1) * gs] = 1.0
            mat_rows.append(row)
            w_rows.append(np.exp(norm[i]) / np.float32(g))
    while len(mat_rows) < 16:
        mat_rows.append(np.zeros(N_BINS, dtype=np.float32))
        w_rows.append(np.float32(0.0))
    return np.stack(mat_rows), np.array(w_rows, dtype=np.float32).reshape(16, 1)


_GROUP_MAT, _GROUP_W = _static_weights()


def _sc_body(feat_hbm, lab_hbm, outsum, outsq, outcnt,
             rows_a, rows_b, lab_v, idx_v, acc_sum, acc_sq, acc_cnt,
             sem_a, sem_b):
    c = lax.axis_index("c")
    s = lax.axis_index("s")
    wid = c * NS + s
    base = wid * ROWS_PER_W
    zero16 = jnp.zeros((LANES,), jnp.float32)
    lane = lax.iota(jnp.int32, LANES)
    one_row = jnp.where(lane == 0, jnp.float32(1.0), jnp.float32(0.0))

    def zrow(i, _):
        for b in range(N_BINS):
            acc_sum[b, pl.ds(i * LANES, LANES)] = zero16
        return 0
    lax.fori_loop(0, N_FEAT // LANES, zrow, 0)
    for b in range(N_BINS):
        acc_sq[b, pl.ds(0, LANES)] = zero16
        acc_cnt[b, pl.ds(0, LANES)] = zero16

    pltpu.sync_copy(lab_hbm.at[pl.ds(base, ROWS_PER_W)], lab_v)
    for g in range(ROWS_PER_W // LANES):
        lv = lab_v[pl.ds(g * LANES, LANES)]
        bi = (lv * jnp.float32(N_BINS)).astype(jnp.int32)
        idx_v[pl.ds(g * LANES, LANES)] = jnp.clip(bi, 0, N_BINS - 1)

    def start(blk, buf, sem):
        pltpu.async_copy(feat_hbm.at[pl.ds(base + blk * BLK, BLK)], buf, sem)

    def wait(blk, buf, sem):
        pltpu.make_async_copy(feat_hbm.at[pl.ds(base + blk * BLK, BLK)],
                              buf, sem).wait()

    ILV = 4

    def accumulate(blk, buf):
        def rown(rr, _):
            r = rr * ILV
            bins = [idx_v[pl.ds(blk * BLK + r + i, LANES)][0]
                    for i in range(ILV)]
            sq = [zero16] * ILV
            for ch in range(N_FEAT // LANES):
                sl = pl.ds(ch * LANES, LANES)
                vs = [buf[r + i, sl] for i in range(ILV)]
                for i in range(ILV):
                    plsc.addupdate(acc_sum.at[bins[i], sl], vs[i])
                for i in range(ILV):
                    sq[i] = sq[i] + vs[i] * vs[i]
            for i in range(ILV):
                plsc.addupdate(acc_sq.at[bins[i], pl.ds(0, LANES)], sq[i])
                plsc.addupdate(acc_cnt.at[bins[i], pl.ds(0, LANES)], one_row)
            return 0

        @plsc.parallel_loop(0, BLK // ILV, 1, unroll=2)
        def _(rr):
            rown(rr, 0)

    start(0, rows_a, sem_a)
    def pipe(h, _):
        b0 = h * 2
        start(b0 + 1, rows_b, sem_b)
        wait(b0, rows_a, sem_a)
        accumulate(b0, rows_a)

        @pl.when(b0 + 2 < NBLK)
        def _():
            start(b0 + 2, rows_a, sem_a)

        wait(b0 + 1, rows_b, sem_b)
        accumulate(b0 + 1, rows_b)
        return 0

    lax.fori_loop(0, NBLK // 2, pipe, 0)

    pltpu.sync_copy(acc_sum, outsum.at[wid])
    pltpu.sync_copy(acc_sq, outsq.at[wid])
    pltpu.sync_copy(acc_cnt, outcnt.at[wid])


def _make_sc_call():
    mesh = plsc.VectorSubcoreMesh(core_axis_name="c", subcore_axis_name="s")
    return pl.kernel(
        _sc_body,
        out_type=[
            jax.ShapeDtypeStruct((NW, N_BINS, N_FEAT), jnp.float32),
            jax.ShapeDtypeStruct((NW, N_BINS, LANES), jnp.float32),
            jax.ShapeDtypeStruct((NW, N_BINS, LANES), jnp.float32),
        ],
        mesh=mesh,
        scratch_types=[
            pltpu.VMEM((BLK, N_FEAT), jnp.float32),
            pltpu.VMEM((BLK, N_FEAT), jnp.float32),
            pltpu.VMEM((ROWS_PER_W,), jnp.float32),
            pltpu.VMEM((ROWS_PER_W + LANES,), jnp.int32),
            pltpu.VMEM((N_BINS, N_FEAT), jnp.float32),
            pltpu.VMEM((N_BINS, LANES), jnp.float32),
            pltpu.VMEM((N_BINS, LANES), jnp.float32),
            pltpu.SemaphoreType.DMA,
            pltpu.SemaphoreType.DMA,
        ],
    )


def _combine_body(sum_ref, sq_ref, cnt_ref, mat_ref, w_ref, out_ref):
    binsum = sum_ref[0]
    binsq = sq_ref[0]
    cnt = cnt_ref[0]
    for w in range(1, NW):
        binsum = binsum + sum_ref[w]
        binsq = binsq + sq_ref[w]
        cnt = cnt + cnt_ref[w]
    q = jnp.sum(binsq, axis=1, keepdims=True)
    c = cnt[:, 0:1]
    m = mat_ref[...]
    sg = jnp.dot(m, binsum, preferred_element_type=jnp.float32)
    qg = jnp.dot(m, q, preferred_element_type=jnp.float32)
    cg = jnp.maximum(jnp.dot(m, c, preferred_element_type=jnp.float32), 1.0)
    nrm = jnp.sum(sg * sg, axis=1, keepdims=True)
    l2 = qg / cg - nrm / (cg * cg)
    out_ref[0, 0] = jnp.sum(l2 * w_ref[...])


def kernel(features, labels):
    sc_call = _make_sc_call()
    psum, psq, pcnt = sc_call(features, labels)
    loss = pl.pallas_call(
        _combine_body,
        out_shape=jax.ShapeDtypeStruct((1, 1), jnp.float32),
        out_specs=pl.BlockSpec(memory_space=pltpu.SMEM),
    )(psum, psq, pcnt, jnp.asarray(_GROUP_MAT), jnp.asarray(_GROUP_W))
    return loss.reshape(1)


# speedup vs baseline: 2.0919x; 1.9353x over previous
import functools

import jax
import jax.numpy as jnp
import numpy as np
from jax import lax
from jax.experimental import pallas as pl
from jax.experimental.pallas import tpu as pltpu
from jax.experimental.pallas import tpu_sc as plsc

N_ROWS = 8192
N_FEAT = 1024
N_BINS = 16
NC = 2
NS = 16
NW = NC * NS
ROWS_PER_W = N_ROWS // NW
BLK = 32
NBLK = ROWS_PER_W // BLK
LANES = 16

_BIN_COUNTS = np.array(
    [900, 1200, 750, 1100, 980, 640, 1300, 870, 1020, 760, 1150, 690, 940,
     1080, 820, 1010], dtype=np.float64)


def _static_weights():
    mat_rows = []
    w_rows = []
    for gs in (2, 4, 8):
        g = N_BINS // gs
        sums = _BIN_COUNTS.reshape(g, gs).sum(axis=1)
        diffs = sums.max() - sums
        md = diffs.max() if diffs.max() != 0 else 1.0
        norm = (diffs / md).astype(np.float32)
        for i in range(g):
            row = np.zeros(N_BINS, dtype=np.float32)
            row[i * gs:(i + 1) * gs] = 1.0
            mat_rows.append(row)
            w_rows.append(np.exp(norm[i]) / np.float32(g))
    while len(mat_rows) < 16:
        mat_rows.append(np.zeros(N_BINS, dtype=np.float32))
        w_rows.append(np.float32(0.0))
    return np.stack(mat_rows), np.array(w_rows, dtype=np.float32).reshape(16, 1)


_GROUP_MAT, _GROUP_W = _static_weights()


def _sc_body(feat_hbm, lab_hbm, outsum, outsq, outcnt,
             rows_a, rows_b, lab_v, idx_v, acc_sum, acc_sq, acc_cnt,
             sem_a, sem_b):
    c = lax.axis_index("c")
    s = lax.axis_index("s")
    wid = c * NS + s
    base = wid * ROWS_PER_W
    zero16 = jnp.zeros((LANES,), jnp.float32)
    lane = lax.iota(jnp.int32, LANES)
    one_row = jnp.where(lane == 0, jnp.float32(1.0), jnp.float32(0.0))

    def zrow(i, _):
        for b in range(N_BINS):
            acc_sum[b, pl.ds(i * LANES, LANES)] = zero16
        return 0
    lax.fori_loop(0, N_FEAT // LANES, zrow, 0)
    for b in range(N_BINS):
        acc_sq[b, pl.ds(0, LANES)] = zero16
        acc_cnt[b, pl.ds(0, LANES)] = zero16

    pltpu.sync_copy(lab_hbm.at[pl.ds(base, ROWS_PER_W)], lab_v)
    for g in range(ROWS_PER_W // LANES):
        lv = lab_v[pl.ds(g * LANES, LANES)]
        bi = (lv * jnp.float32(N_BINS)).astype(jnp.int32)
        idx_v[pl.ds(g * LANES, LANES)] = jnp.clip(bi, 0, N_BINS - 1)

    def start(blk, buf, sem):
        pltpu.async_copy(feat_hbm.at[pl.ds(base + blk * BLK, BLK)], buf, sem)

    def wait(blk, buf, sem):
        pltpu.make_async_copy(feat_hbm.at[pl.ds(base + blk * BLK, BLK)],
                              buf, sem).wait()

    ILV = 4
    SEG = 16

    def accumulate(blk, buf):
        def rown(rr, _):
            r = rr * ILV
            bins = [idx_v[pl.ds(blk * BLK + r + i, LANES)][0]
                    for i in range(ILV)]

            def seg(si, sq):
                sq = list(sq)
                for chl in range(SEG):
                    sl = pl.ds(si * (SEG * LANES) + chl * LANES, LANES)
                    vs = [buf[r + i, sl] for i in range(ILV)]
                    for i in range(ILV):
                        plsc.addupdate(acc_sum.at[bins[i], sl], vs[i])
                    for i in range(ILV):
                        sq[i] = sq[i] + vs[i] * vs[i]
                return tuple(sq)

            sq = lax.fori_loop(0, N_FEAT // LANES // SEG, seg,
                               (zero16,) * ILV)
            for i in range(ILV):
                plsc.addupdate(acc_sq.at[bins[i], pl.ds(0, LANES)], sq[i])
                plsc.addupdate(acc_cnt.at[bins[i], pl.ds(0, LANES)], one_row)
            return 0

        lax.fori_loop(0, BLK // ILV, rown, 0)

    start(0, rows_a, sem_a)
    def pipe(h, _):
        b0 = h * 2
        start(b0 + 1, rows_b, sem_b)
        wait(b0, rows_a, sem_a)
        accumulate(b0, rows_a)

        @pl.when(b0 + 2 < NBLK)
        def _():
            start(b0 + 2, rows_a, sem_a)

        wait(b0 + 1, rows_b, sem_b)
        accumulate(b0 + 1, rows_b)
        return 0

    lax.fori_loop(0, NBLK // 2, pipe, 0)

    pltpu.sync_copy(acc_sum, outsum.at[wid])
    pltpu.sync_copy(acc_sq, outsq.at[wid])
    pltpu.sync_copy(acc_cnt, outcnt.at[wid])


def _make_sc_call():
    mesh = plsc.VectorSubcoreMesh(core_axis_name="c", subcore_axis_name="s")
    return pl.kernel(
        _sc_body,
        out_type=[
            jax.ShapeDtypeStruct((NW, N_BINS, N_FEAT), jnp.float32),
            jax.ShapeDtypeStruct((NW, N_BINS, LANES), jnp.float32),
            jax.ShapeDtypeStruct((NW, N_BINS, LANES), jnp.float32),
        ],
        mesh=mesh,
        scratch_types=[
            pltpu.VMEM((BLK, N_FEAT), jnp.float32),
            pltpu.VMEM((BLK, N_FEAT), jnp.float32),
            pltpu.VMEM((ROWS_PER_W,), jnp.float32),
            pltpu.VMEM((ROWS_PER_W + LANES,), jnp.int32),
            pltpu.VMEM((N_BINS, N_FEAT), jnp.float32),
            pltpu.VMEM((N_BINS, LANES), jnp.float32),
            pltpu.VMEM((N_BINS, LANES), jnp.float32),
            pltpu.SemaphoreType.DMA,
            pltpu.SemaphoreType.DMA,
        ],
    )


def _combine_body(sum_ref, sq_ref, cnt_ref, mat_ref, w_ref, out_ref):
    binsum = sum_ref[0]
    binsq = sq_ref[0]
    cnt = cnt_ref[0]
    for w in range(1, NW):
        binsum = binsum + sum_ref[w]
        binsq = binsq + sq_ref[w]
        cnt = cnt + cnt_ref[w]
    q = jnp.sum(binsq, axis=1, keepdims=True)
    c = cnt[:, 0:1]
    m = mat_ref[...]
    sg = jnp.dot(m, binsum, preferred_element_type=jnp.float32)
    qg = jnp.dot(m, q, preferred_element_type=jnp.float32)
    cg = jnp.maximum(jnp.dot(m, c, preferred_element_type=jnp.float32), 1.0)
    nrm = jnp.sum(sg * sg, axis=1, keepdims=True)
    l2 = qg / cg - nrm / (cg * cg)
    out_ref[0, 0] = jnp.sum(l2 * w_ref[...])


def kernel(features, labels):
    sc_call = _make_sc_call()
    psum, psq, pcnt = sc_call(features, labels)
    loss = pl.pallas_call(
        _combine_body,
        out_shape=jax.ShapeDtypeStruct((1, 1), jnp.float32),
        out_specs=pl.BlockSpec(memory_space=pltpu.SMEM),
    )(psum, psq, pcnt, jnp.asarray(_GROUP_MAT), jnp.asarray(_GROUP_W))
    return loss.reshape(1)
